# baseline (speedup 1.0000x reference)
import jax
import jax.numpy as jnp
from jax.experimental import pallas as pl
from jax.experimental.pallas import tpu as pltpu

_B = 16
_S = 4096
_D = 1024
_PE = 64
_N = 32768
_BS = 512


def _pe_add_kernel(seg_ref, pe_ref, x_ref, o_ref):
    b = pl.program_id(0)
    s = pl.program_id(1)
    p0 = s * _BS
    seg = seg_ref[...]
    start_b = jnp.sum((seg < b).astype(jnp.int32))
    len_b = jnp.sum((seg == b).astype(jnp.int32))

    @pl.when(p0 > len_b)
    def _copy():
        o_ref[...] = x_ref[...]

    @pl.when(p0 <= len_b)
    def _add():
        j0 = start_b + p0
        pe_blk = pe_ref[pl.ds(j0, _BS), :]
        p = p0 + jax.lax.broadcasted_iota(jnp.int32, (_BS, 1), 0)
        valid = (p >= 1) & (p <= len_b)
        pe_blk = jnp.where(valid, pe_blk, 0.0)
        last = pe_blk[:, _PE - 1:_PE]
        pe128 = jnp.concatenate(
            [pe_blk, jnp.broadcast_to(last, (_BS, 128 - _PE))], axis=1
        )
        xb = x_ref[0]
        o_ref[0, :, :128] = xb[:, :128] + pe128
        o_ref[0, :, 128:] = xb[:, 128:] + last

    del _copy, _add


def kernel(x, flat_pe, segment_ids):
    pe_ext = jnp.concatenate(
        [jnp.zeros((1, _PE), x.dtype), flat_pe, jnp.zeros((_S - 1, _PE), x.dtype)]
    )
    seg2d = segment_ids.reshape(8, _N // 8)
    grid = (_B, _S // _BS)
    return pl.pallas_call(
        _pe_add_kernel,
        grid=grid,
        in_specs=[
            pl.BlockSpec((8, _N // 8), lambda b, s: (0, 0)),
            pl.BlockSpec((_N + _S, _PE), lambda b, s: (0, 0)),
            pl.BlockSpec((1, _BS, _D), lambda b, s: (b, s, 0)),
        ],
        out_specs=pl.BlockSpec((1, _BS, _D), lambda b, s: (b, s, 0)),
        out_shape=jax.ShapeDtypeStruct(x.shape, x.dtype),
        compiler_params=pltpu.CompilerParams(
            dimension_semantics=("parallel", "parallel"),
        ),
    )(seg2d, pe_ext, x)


# speedup vs baseline: 1.0835x; 1.0835x over previous
import jax
import jax.numpy as jnp
from jax.experimental import pallas as pl
from jax.experimental.pallas import tpu as pltpu

_B = 16
_S = 4096
_D = 1024
_PE = 64
_N = 32768
_BS = 1024


def _pe_add_kernel(seg_hbm, pe_hbm, x_ref, o_ref, seg_v, pe_v, sem_seg, sem_pe):
    b = pl.program_id(0)
    s = pl.program_id(1)
    p0 = s * _BS

    @pl.when((b == 0) & (s == 0))
    def _load_once():
        pltpu.make_async_copy(seg_hbm, seg_v, sem_seg).start()
        pltpu.make_async_copy(pe_hbm, pe_v, sem_pe).start()
        pltpu.make_async_copy(seg_hbm, seg_v, sem_seg).wait()
        pltpu.make_async_copy(pe_hbm, pe_v, sem_pe).wait()

    seg = seg_v[...]
    start_b = jnp.sum((seg < b).astype(jnp.int32))
    len_b = jnp.sum((seg == b).astype(jnp.int32))

    @pl.when(p0 > len_b)
    def _copy():
        o_ref[...] = x_ref[...]

    @pl.when(p0 <= len_b)
    def _add():
        j0 = start_b + p0
        pe_blk = pe_v[pl.ds(j0, _BS), :]
        p = p0 + jax.lax.broadcasted_iota(jnp.int32, (_BS, 1), 0)
        valid = (p >= 1) & (p <= len_b)
        pe_blk = jnp.where(valid, pe_blk, 0.0)
        last = pe_blk[:, _PE - 1:_PE]
        pe128 = jnp.concatenate(
            [pe_blk, jnp.broadcast_to(last, (_BS, 128 - _PE))], axis=1
        )
        xb = x_ref[0]
        o_ref[0, :, :128] = xb[:, :128] + pe128
        o_ref[0, :, 128:] = xb[:, 128:] + last

    del _load_once, _copy, _add


def kernel(x, flat_pe, segment_ids):
    pe_ext = jnp.concatenate(
        [jnp.zeros((1, _PE), x.dtype), flat_pe, jnp.zeros((_S - 1, _PE), x.dtype)]
    )
    seg2d = segment_ids.reshape(8, _N // 8)
    grid = (_B, _S // _BS)
    return pl.pallas_call(
        _pe_add_kernel,
        grid=grid,
        in_specs=[
            pl.BlockSpec(memory_space=pltpu.MemorySpace.HBM),
            pl.BlockSpec(memory_space=pltpu.MemorySpace.HBM),
            pl.BlockSpec((1, _BS, _D), lambda b, s: (b, s, 0)),
        ],
        out_specs=pl.BlockSpec((1, _BS, _D), lambda b, s: (b, s, 0)),
        out_shape=jax.ShapeDtypeStruct(x.shape, x.dtype),
        scratch_shapes=[
            pltpu.VMEM((8, _N // 8), jnp.int32),
            pltpu.VMEM((_N + _S, _PE), jnp.float32),
            pltpu.SemaphoreType.DMA,
            pltpu.SemaphoreType.DMA,
        ],
        compiler_params=pltpu.CompilerParams(
            dimension_semantics=("arbitrary", "arbitrary"),
        ),
    )(seg2d, pe_ext, x)


# speedup vs baseline: 1.4237x; 1.3140x over previous
import jax
import jax.numpy as jnp
from jax.experimental import pallas as pl
from jax.experimental.pallas import tpu as pltpu

_B = 16
_S = 4096
_D = 1024
_PE = 64
_N = 32768
_BS = 1024


def _pe_add_kernel(seg_hbm, pe_hbm, x_ref, o_ref, seg_v, pe_v, sem_seg, sem_pe):
    b = pl.program_id(0)
    s = pl.program_id(1)
    p0 = s * _BS

    @pl.when((b == 0) & (s == 0))
    def _load_once():
        pltpu.make_async_copy(seg_hbm, seg_v, sem_seg).start()
        pltpu.make_async_copy(pe_hbm, pe_v.at[pl.ds(1, _N), :], sem_pe).start()
        pltpu.make_async_copy(seg_hbm, seg_v, sem_seg).wait()
        pltpu.make_async_copy(pe_hbm, pe_v.at[pl.ds(1, _N), :], sem_pe).wait()

    seg = seg_v[...]
    start_b = jnp.sum((seg < b).astype(jnp.int32))
    len_b = jnp.sum((seg == b).astype(jnp.int32))

    @pl.when(p0 > len_b)
    def _copy():
        o_ref[...] = x_ref[...]

    @pl.when(p0 <= len_b)
    def _add():
        j0 = start_b + p0
        pe_blk = pe_v[pl.ds(j0, _BS), :]
        p = p0 + jax.lax.broadcasted_iota(jnp.int32, (_BS, 1), 0)
        valid = (p >= 1) & (p <= len_b)
        pe_blk = jnp.where(valid, pe_blk, 0.0)
        last = pe_blk[:, _PE - 1:_PE]
        pe128 = jnp.concatenate(
            [pe_blk, jnp.broadcast_to(last, (_BS, 128 - _PE))], axis=1
        )
        xb = x_ref[0]
        o_ref[0, :, :128] = xb[:, :128] + pe128
        o_ref[0, :, 128:] = xb[:, 128:] + last

    del _load_once, _copy, _add


def kernel(x, flat_pe, segment_ids):
    seg2d = segment_ids.reshape(8, _N // 8)
    grid = (_B, _S // _BS)
    return pl.pallas_call(
        _pe_add_kernel,
        grid=grid,
        in_specs=[
            pl.BlockSpec(memory_space=pltpu.MemorySpace.HBM),
            pl.BlockSpec(memory_space=pltpu.MemorySpace.HBM),
            pl.BlockSpec((1, _BS, _D), lambda b, s: (b, s, 0)),
        ],
        out_specs=pl.BlockSpec((1, _BS, _D), lambda b, s: (b, s, 0)),
        out_shape=jax.ShapeDtypeStruct(x.shape, x.dtype),
        scratch_shapes=[
            pltpu.VMEM((8, _N // 8), jnp.int32),
            pltpu.VMEM((_N + _S, _PE), jnp.float32),
            pltpu.SemaphoreType.DMA,
            pltpu.SemaphoreType.DMA,
        ],
        compiler_params=pltpu.CompilerParams(
            dimension_semantics=("arbitrary", "arbitrary"),
        ),
    )(seg2d, flat_pe, x)
